# Initial kernel scaffold; baseline (speedup 1.0000x reference)
import jax
import jax.numpy as jnp
from jax.experimental import pallas as pl


def kernel(x, edge_index, doc_feature, pool_w, gru_w_ih, gru_w_hh, gru_b_ih, gru_b_hh, init_w, gnn_fc_w, gnn_fc_b, doc_fc_w, doc_fc_b, ln_g, ln_b, fusion_w, fusion_b, task_w, task_b, time_w, time_b):
    raise NotImplementedError("write your pallas kernel here")


# speedup vs baseline: 69.4542x; 69.4542x over previous
import functools

import jax
import jax.numpy as jnp
from jax import lax
from jax.experimental import pallas as pl
from jax.experimental.pallas import tpu as pltpu
from jax.experimental.pallas import tpu_sc as plsc

_N = 10000
_E = 320000
_P = 10240
_R = _P // 128
_NSUB = 16
_EPT = _E // _NSUB
_NEV = _EPT // 16
_SLAB = _P // _NSUB
_NSV = _SLAB // 16


def _fast_rsqrt(t):
    i = plsc.bitcast(t, jnp.int32)
    i = jnp.int32(0x5F3759DF) - lax.shift_right_logical(i, 1)
    y = plsc.bitcast(i, jnp.float32)
    for _ in range(3):
        y = y * (1.5 - 0.5 * t * y * y)
    return y


def _sc_coeff(src_i, dst_i, zeros_p):
    mesh = plsc.VectorSubcoreMesh(
        core_axis_name="c", subcore_axis_name="s",
        num_cores=1, num_subcores=_NSUB)

    @functools.partial(
        pl.kernel,
        out_type=jax.ShapeDtypeStruct((_P,), jnp.float32),
        mesh=mesh,
        scratch_types=[
            pltpu.VMEM((_EPT,), jnp.int32),
            pltpu.VMEM((_EPT,), jnp.int32),
            pltpu.VMEM((_P,), jnp.float32),
            pltpu.VMEM((_P,), jnp.float32),
            pltpu.VMEM((_NSUB, _SLAB), jnp.float32),
            pltpu.VMEM((_SLAB,), jnp.float32),
            pltpu.VMEM_SHARED((_NSUB, _P), jnp.float32),
            pltpu.VMEM_SHARED((_P,), jnp.float32),
        ],
        compiler_params=pltpu.CompilerParams(needs_layout_passes=False),
    )
    def k(src_hbm, dst_hbm, zero_hbm, coeff_hbm,
          dst_v, src_v, acc_v, dinv_v, slab_v, co_v, part_sh, dinv_sh):
        sid = lax.axis_index("s")
        base = sid * _EPT
        pltpu.sync_copy(dst_hbm.at[pl.ds(base, _EPT)], dst_v)
        pltpu.sync_copy(src_hbm.at[pl.ds(base, _EPT)], src_v)
        pltpu.sync_copy(zero_hbm, acc_v)
        ones = jnp.ones((16,), jnp.float32)

        def p1(i, c):
            idx = dst_v[pl.ds(i * 16, 16)]
            plsc.addupdate_scatter(acc_v, [idx], ones)
            return c
        lax.fori_loop(0, _NEV, p1, 0)

        pltpu.sync_copy(acc_v, part_sh.at[sid])
        plsc.subcore_barrier()
        pltpu.sync_copy(part_sh.at[:, pl.ds(sid * _SLAB, _SLAB)], slab_v)

        def rd(j, c):
            t = jnp.ones((16,), jnp.float32)
            for r in range(_NSUB):
                t = t + slab_v[r, pl.ds(j * 16, 16)]
            co_v[pl.ds(j * 16, 16)] = _fast_rsqrt(t)
            return c
        lax.fori_loop(0, _NSV, rd, 0)

        pltpu.sync_copy(co_v, dinv_sh.at[pl.ds(sid * _SLAB, _SLAB)])
        plsc.subcore_barrier()
        pltpu.sync_copy(dinv_sh, dinv_v)
        pltpu.sync_copy(zero_hbm, acc_v)

        def p2(i, c):
            di = dst_v[pl.ds(i * 16, 16)]
            vals = plsc.load_gather(dinv_v, [di])
            si = src_v[pl.ds(i * 16, 16)]
            plsc.addupdate_scatter(acc_v, [si], vals)
            return c
        lax.fori_loop(0, _NEV, p2, 0)

        pltpu.sync_copy(acc_v, part_sh.at[sid])
        plsc.subcore_barrier()
        pltpu.sync_copy(part_sh.at[:, pl.ds(sid * _SLAB, _SLAB)], slab_v)

        def rc(j, c):
            t = jnp.zeros((16,), jnp.float32)
            for r in range(_NSUB):
                t = t + slab_v[r, pl.ds(j * 16, 16)]
            d = dinv_v[pl.ds(sid * _SLAB + j * 16, 16)]
            co_v[pl.ds(j * 16, 16)] = d * (t + d)
            return c
        lax.fori_loop(0, _NSV, rc, 0)
        pltpu.sync_copy(co_v, coeff_hbm.at[pl.ds(sid * _SLAB, _SLAB)])

    return k(src_i, dst_i, zeros_p)


def _tc_evolve(x3t, x_pad, pool_w2, wih_t, whh_t, bih2, bhh2, init_w):
    def body(x3t_ref, xp_ref, pw_ref, wih_ref, whh_ref, bih_ref, bhh_ref,
             h_ref, w_out_ref, sc_ref, xt_ref):
        pw = pw_ref[...]
        inv = 1.0 / jnp.sqrt(jnp.sum(pw * pw))

        def sc_loop(r, c):
            chunk = x3t_ref[r]
            srow = jnp.dot(pw, chunk, preferred_element_type=jnp.float32)
            sc_ref[pl.ds(r, 1), :] = jnp.tanh(srow * inv)
            return c
        lax.fori_loop(0, _R, sc_loop, 0)

        rows = lax.broadcasted_iota(jnp.int32, (_R, 128), 0)
        cols = lax.broadcasted_iota(jnp.int32, (_R, 128), 1)
        flat = rows * 128 + cols
        sc_ref[...] = jnp.where(flat < _N, sc_ref[...], -2.0)

        def pick(i, c):
            sw = sc_ref[...]
            m = jnp.max(sw)
            idx = jnp.min(jnp.where(sw == m, flat, jnp.int32(_P)))
            row = xp_ref[pl.ds(idx, 1), :]
            xt_ref[pl.ds(i, 1), :] = row * m
            sc_ref[...] = jnp.where(flat == idx, -3.0, sw)
            return c
        lax.fori_loop(0, 128, pick, 0)

        xt = xt_ref[...]
        gx = jnp.dot(xt, wih_ref[...],
                     preferred_element_type=jnp.float32) + bih_ref[...]
        h = h_ref[...]
        gh = jnp.dot(h, whh_ref[...],
                     preferred_element_type=jnp.float32) + bhh_ref[...]
        xr, xz, xn = gx[:, :128], gx[:, 128:256], gx[:, 256:]
        hr, hz, hn = gh[:, :128], gh[:, 128:256], gh[:, 256:]
        r_ = jax.nn.sigmoid(xr + hr)
        z_ = jax.nn.sigmoid(xz + hz)
        ng = jnp.tanh(xn + r_ * hn)
        w_out_ref[...] = (1.0 - z_) * ng + z_ * h

    return pl.pallas_call(
        body,
        out_shape=jax.ShapeDtypeStruct((128, 128), jnp.float32),
        scratch_shapes=[
            pltpu.VMEM((_R, 128), jnp.float32),
            pltpu.VMEM((128, 128), jnp.float32),
        ],
    )(x3t, x_pad, pool_w2, wih_t, whh_t, bih2, bhh2, init_w)


def _tc_head(coeff2, x_pad, w_g, gw_t, gb2, doc2, dw_t, db2, lg2, lb2,
             fw_t, fb2, tw_t, tb2, mw_t, mb2):
    def body(c_ref, xp_ref, w_ref, gw_ref, gb_ref, d_ref, dw_ref, db_ref,
             lg_ref, lb_ref, fw_ref, fb_ref, tw_ref, tb_ref, mw_ref, mb_ref,
             o1_ref, o2_ref):
        cx = jnp.dot(c_ref[...], xp_ref[...],
                     preferred_element_type=jnp.float32)
        pooled = jnp.dot(cx, w_ref[...],
                         preferred_element_type=jnp.float32) * (1.0 / _N)
        x_g = jnp.dot(pooled, gw_ref[...],
                      preferred_element_type=jnp.float32) + gb_ref[...]
        dq = jnp.maximum(
            jnp.dot(d_ref[...], dw_ref[...],
                    preferred_element_type=jnp.float32) + db_ref[...], 0.0)
        mu = (jnp.sum(x_g) + jnp.sum(dq)) / 512.0
        var = (jnp.sum((x_g - mu) ** 2) + jnp.sum((dq - mu) ** 2)) / 512.0
        isd = lax.rsqrt(var + 1e-5)
        a = (x_g - mu) * isd * lg_ref[:, :256] + lb_ref[:, :256]
        b = (dq - mu) * isd * lg_ref[:, 256:] + lb_ref[:, 256:]
        fused = (jnp.dot(a, fw_ref[:256, :], preferred_element_type=jnp.float32)
                 + jnp.dot(b, fw_ref[256:, :], preferred_element_type=jnp.float32)
                 + fb_ref[...])
        hh = jnp.maximum(fused, 0.0)
        o1_ref[...] = jnp.dot(hh, tw_ref[...],
                              preferred_element_type=jnp.float32) + tb_ref[...]
        o2_ref[...] = jnp.dot(hh, mw_ref[...],
                              preferred_element_type=jnp.float32) + mb_ref[...]

    return pl.pallas_call(
        body,
        out_shape=(jax.ShapeDtypeStruct((1, 10), jnp.float32),
                   jax.ShapeDtypeStruct((1, 1), jnp.float32)),
    )(coeff2, x_pad, w_g, gw_t, gb2, doc2, dw_t, db2, lg2, lb2,
      fw_t, fb2, tw_t, tb2, mw_t, mb2)


def kernel(x, edge_index, doc_feature, pool_w, gru_w_ih, gru_w_hh, gru_b_ih,
           gru_b_hh, init_w, gnn_fc_w, gnn_fc_b, doc_fc_w, doc_fc_b, ln_g,
           ln_b, fusion_w, fusion_b, task_w, task_b, time_w, time_b):
    src = edge_index[0]
    dst = edge_index[1]
    zeros_p = jnp.zeros((_P,), jnp.float32)
    coeff = _sc_coeff(src, dst, zeros_p)

    x_pad = jnp.concatenate(
        [x, jnp.zeros((_P - _N, 128), jnp.float32)], axis=0)
    x3t = x_pad.reshape(_R, 128, 128).transpose(0, 2, 1)

    w_g = _tc_evolve(
        x3t, x_pad, pool_w.reshape(1, 128), gru_w_ih.T, gru_w_hh.T,
        gru_b_ih.reshape(1, -1), gru_b_hh.reshape(1, -1), init_w)

    return _tc_head(
        coeff.reshape(1, _P), x_pad, w_g, gnn_fc_w.T,
        gnn_fc_b.reshape(1, -1), doc_feature.reshape(1, -1), doc_fc_w.T,
        doc_fc_b.reshape(1, -1), ln_g.reshape(1, -1), ln_b.reshape(1, -1),
        fusion_w.T, fusion_b.reshape(1, -1), task_w.T,
        task_b.reshape(1, -1), time_w.T, time_b.reshape(1, -1))


# speedup vs baseline: 72.8990x; 1.0496x over previous
import functools

import jax
import jax.numpy as jnp
from jax import lax
from jax.experimental import pallas as pl
from jax.experimental.pallas import tpu as pltpu
from jax.experimental.pallas import tpu_sc as plsc

_N = 10000
_E = 320000
_P = 10240
_NCORE = 2
_NSUB = 16
_EPT = _E // _NSUB
_NEV = _EPT // 16
_EPT2 = _E // (_NCORE * _NSUB)
_NEV2 = _EPT2 // 16
_SLAB = _P // _NSUB
_NSV = _SLAB // 16
_RF = 78
_RS = 79


def _fast_rsqrt(t):
    i = plsc.bitcast(t, jnp.int32)
    i = jnp.int32(0x5F3759DF) - lax.shift_right_logical(i, 1)
    y = plsc.bitcast(i, jnp.float32)
    for _ in range(3):
        y = y * (1.5 - 0.5 * t * y * y)
    return y


def _sc_edge(src_i, dst_i, zeros_p):
    mesh = plsc.VectorSubcoreMesh(
        core_axis_name="c", subcore_axis_name="s",
        num_cores=_NCORE, num_subcores=_NSUB)

    @functools.partial(
        pl.kernel,
        out_type=(jax.ShapeDtypeStruct((_P,), jnp.float32),
                  jax.ShapeDtypeStruct((_P,), jnp.float32),
                  jax.ShapeDtypeStruct((_P,), jnp.float32)),
        mesh=mesh,
        scratch_types=[
            pltpu.VMEM((_EPT,), jnp.int32),
            pltpu.VMEM((_EPT2,), jnp.int32),
            pltpu.VMEM((_P,), jnp.float32),
            pltpu.VMEM((_P,), jnp.float32),
            pltpu.VMEM((_NSUB, _SLAB), jnp.float32),
            pltpu.VMEM((_SLAB,), jnp.float32),
            pltpu.VMEM_SHARED((_NSUB, _P), jnp.float32),
            pltpu.VMEM_SHARED((_P,), jnp.float32),
        ],
        compiler_params=pltpu.CompilerParams(needs_layout_passes=False),
    )
    def k(src_hbm, dst_hbm, zero_hbm, dinv_hbm, s0_hbm, s1_hbm,
          dst_v, src2_v, acc_v, dinv_v, slab_v, co_v, part_sh, dinv_sh):
        cid = lax.axis_index("c")
        sid = lax.axis_index("s")
        base = sid * _EPT
        pltpu.sync_copy(dst_hbm.at[pl.ds(base, _EPT)], dst_v)
        pltpu.sync_copy(zero_hbm, acc_v)
        ones = jnp.ones((16,), jnp.float32)

        def p1(i, c):
            idx = dst_v[pl.ds(i * 16, 16)]
            plsc.addupdate_scatter(acc_v, [idx], ones)
            return c
        lax.fori_loop(0, _NEV, p1, 0, unroll=8)

        pltpu.sync_copy(acc_v, part_sh.at[sid])
        plsc.subcore_barrier()
        pltpu.sync_copy(part_sh.at[:, pl.ds(sid * _SLAB, _SLAB)], slab_v)

        def rd(j, c):
            t = jnp.ones((16,), jnp.float32)
            for r in range(_NSUB):
                t = t + slab_v[r, pl.ds(j * 16, 16)]
            co_v[pl.ds(j * 16, 16)] = _fast_rsqrt(t)
            return c
        lax.fori_loop(0, _NSV, rd, 0)

        pltpu.sync_copy(co_v, dinv_sh.at[pl.ds(sid * _SLAB, _SLAB)])

        @pl.when(cid == 0)
        def _():
            pltpu.sync_copy(co_v, dinv_hbm.at[pl.ds(sid * _SLAB, _SLAB)])

        plsc.subcore_barrier()
        pltpu.sync_copy(dinv_sh, dinv_v)
        pltpu.sync_copy(zero_hbm, acc_v)

        half = cid * _EPT2
        pltpu.sync_copy(src_hbm.at[pl.ds(base + half, _EPT2)], src2_v)

        def p2(i, c):
            di = dst_v[pl.ds(half + i * 16, 16)]
            vals = plsc.load_gather(dinv_v, [di])
            si = src2_v[pl.ds(i * 16, 16)]
            plsc.addupdate_scatter(acc_v, [si], vals)
            return c
        lax.fori_loop(0, _NEV2, p2, 0, unroll=8)

        pltpu.sync_copy(acc_v, part_sh.at[sid])
        plsc.subcore_barrier()
        pltpu.sync_copy(part_sh.at[:, pl.ds(sid * _SLAB, _SLAB)], slab_v)

        def rs(j, c):
            t = jnp.zeros((16,), jnp.float32)
            for r in range(_NSUB):
                t = t + slab_v[r, pl.ds(j * 16, 16)]
            co_v[pl.ds(j * 16, 16)] = t
            return c
        lax.fori_loop(0, _NSV, rs, 0)

        @pl.when(cid == 0)
        def _():
            pltpu.sync_copy(co_v, s0_hbm.at[pl.ds(sid * _SLAB, _SLAB)])

        @pl.when(cid == 1)
        def _():
            pltpu.sync_copy(co_v, s1_hbm.at[pl.ds(sid * _SLAB, _SLAB)])

    return k(src_i, dst_i, zeros_p)


def _tc_evolve(x, pool_w2, wih, whh, bih2, bhh2, init_w):
    def body(x_ref, pw_ref, wih_ref, whh_ref, bih_ref, bhh_ref,
             h_ref, w_out_ref, sc_ref, xt_ref):
        pw = pw_ref[...]
        inv = 1.0 / jnp.sqrt(jnp.sum(pw * pw))

        def sc_loop(r, c):
            chunk = x_ref[pl.ds(r * 128, 128), :]
            srow = lax.dot_general(
                pw, chunk, (((1,), (1,)), ((), ())),
                preferred_element_type=jnp.float32)
            sc_ref[pl.ds(r, 1), :] = jnp.tanh(srow * inv)
            return c
        lax.fori_loop(0, _RF, sc_loop, 0)

        tail = x_ref[pl.ds(_RF * 128, 16), :]
        st = lax.dot_general(pw, tail, (((1,), (1,)), ((), ())),
                             preferred_element_type=jnp.float32)
        sc_ref[pl.ds(_RF, 1), :] = jnp.concatenate(
            [jnp.tanh(st * inv), jnp.full((1, 112), -2.0, jnp.float32)],
            axis=1)

        rows = lax.broadcasted_iota(jnp.int32, (_RS, 128), 0)
        cols = lax.broadcasted_iota(jnp.int32, (_RS, 128), 1)
        flat = rows * 128 + cols

        def pick(i, c):
            sw = sc_ref[...]
            m = jnp.max(sw)
            idx = jnp.min(jnp.where(sw == m, flat, jnp.int32(_P)))
            row = x_ref[pl.ds(idx, 1), :]
            xt_ref[pl.ds(i, 1), :] = row * m
            sc_ref[...] = jnp.where(flat == idx, -3.0, sw)
            return c
        lax.fori_loop(0, 128, pick, 0)

        xt = xt_ref[...]
        gx = lax.dot_general(xt, wih_ref[...], (((1,), (1,)), ((), ())),
                             preferred_element_type=jnp.float32) + bih_ref[...]
        h = h_ref[...]
        gh = lax.dot_general(h, whh_ref[...], (((1,), (1,)), ((), ())),
                             preferred_element_type=jnp.float32) + bhh_ref[...]
        xr, xz, xn = gx[:, :128], gx[:, 128:256], gx[:, 256:]
        hr, hz, hn = gh[:, :128], gh[:, 128:256], gh[:, 256:]
        r_ = jax.nn.sigmoid(xr + hr)
        z_ = jax.nn.sigmoid(xz + hz)
        ng = jnp.tanh(xn + r_ * hn)
        w_out_ref[...] = (1.0 - z_) * ng + z_ * h

    return pl.pallas_call(
        body,
        out_shape=jax.ShapeDtypeStruct((128, 128), jnp.float32),
        scratch_shapes=[
            pltpu.VMEM((_RS, 128), jnp.float32),
            pltpu.VMEM((128, 128), jnp.float32),
        ],
    )(x, pool_w2, wih, whh, bih2, bhh2, init_w)


def _tc_head(dinv2, s2a, s2b, x_pad, w_g, gw_t, gb2, doc2, dw_t, db2,
             lg2, lb2, fw_t, fb2, tw_t, tb2, mw_t, mb2):
    def body(di_ref, sa_ref, sb_ref, xp_ref, w_ref, gw_ref, gb_ref, d_ref,
             dw_ref, db_ref, lg_ref, lb_ref, fw_ref, fb_ref, tw_ref, tb_ref,
             mw_ref, mb_ref, o1_ref, o2_ref):
        dv = di_ref[...]
        coeff = dv * (sa_ref[...] + sb_ref[...] + dv)
        cx = jnp.dot(coeff, xp_ref[...],
                     preferred_element_type=jnp.float32)
        pooled = jnp.dot(cx, w_ref[...],
                         preferred_element_type=jnp.float32) * (1.0 / _N)
        x_g = jnp.dot(pooled, gw_ref[...],
                      preferred_element_type=jnp.float32) + gb_ref[...]
        dq = jnp.maximum(
            jnp.dot(d_ref[...], dw_ref[...],
                    preferred_element_type=jnp.float32) + db_ref[...], 0.0)
        mu = (jnp.sum(x_g) + jnp.sum(dq)) / 512.0
        var = (jnp.sum((x_g - mu) ** 2) + jnp.sum((dq - mu) ** 2)) / 512.0
        isd = lax.rsqrt(var + 1e-5)
        a = (x_g - mu) * isd * lg_ref[:, :256] + lb_ref[:, :256]
        b = (dq - mu) * isd * lg_ref[:, 256:] + lb_ref[:, 256:]
        fused = (jnp.dot(a, fw_ref[:256, :], preferred_element_type=jnp.float32)
                 + jnp.dot(b, fw_ref[256:, :], preferred_element_type=jnp.float32)
                 + fb_ref[...])
        hh = jnp.maximum(fused, 0.0)
        o1_ref[...] = jnp.dot(hh, tw_ref[...],
                              preferred_element_type=jnp.float32) + tb_ref[...]
        o2_ref[...] = jnp.dot(hh, mw_ref[...],
                              preferred_element_type=jnp.float32) + mb_ref[...]

    return pl.pallas_call(
        body,
        out_shape=(jax.ShapeDtypeStruct((1, 10), jnp.float32),
                   jax.ShapeDtypeStruct((1, 1), jnp.float32)),
    )(dinv2, s2a, s2b, x_pad, w_g, gw_t, gb2, doc2, dw_t, db2,
      lg2, lb2, fw_t, fb2, tw_t, tb2, mw_t, mb2)


def kernel(x, edge_index, doc_feature, pool_w, gru_w_ih, gru_w_hh, gru_b_ih,
           gru_b_hh, init_w, gnn_fc_w, gnn_fc_b, doc_fc_w, doc_fc_b, ln_g,
           ln_b, fusion_w, fusion_b, task_w, task_b, time_w, time_b):
    zeros_p = jnp.zeros((_P,), jnp.float32)
    dinv, s0, s1 = _sc_edge(edge_index[0], edge_index[1], zeros_p)

    w_g = _tc_evolve(
        x, pool_w.reshape(1, 128), gru_w_ih, gru_w_hh,
        gru_b_ih.reshape(1, -1), gru_b_hh.reshape(1, -1), init_w)

    x_pad = jnp.concatenate(
        [x, jnp.zeros((_P - _N, 128), jnp.float32)], axis=0)

    return _tc_head(
        dinv.reshape(1, _P), s0.reshape(1, _P), s1.reshape(1, _P),
        x_pad, w_g, gnn_fc_w.T, gnn_fc_b.reshape(1, -1),
        doc_feature.reshape(1, -1), doc_fc_w.T, doc_fc_b.reshape(1, -1),
        ln_g.reshape(1, -1), ln_b.reshape(1, -1), fusion_w.T,
        fusion_b.reshape(1, -1), task_w.T, task_b.reshape(1, -1),
        time_w.T, time_b.reshape(1, -1))


# speedup vs baseline: 80.9932x; 1.1110x over previous
import functools

import jax
import jax.numpy as jnp
from jax import lax
from jax.experimental import pallas as pl
from jax.experimental.pallas import tpu as pltpu
from jax.experimental.pallas import tpu_sc as plsc

_N = 10000
_E = 320000
_P = 10240
_NCORE = 2
_NSUB = 16
_EPT = _E // _NSUB
_NEV = _EPT // 16
_EPT2 = _E // (_NCORE * _NSUB)
_NEV2 = _EPT2 // 16
_SLAB = _P // _NSUB
_NSV = _SLAB // 16
_RF = 78
_RS = 79


def _fast_rsqrt(t):
    i = plsc.bitcast(t, jnp.int32)
    i = jnp.int32(0x5F3759DF) - lax.shift_right_logical(i, 1)
    y = plsc.bitcast(i, jnp.float32)
    for _ in range(3):
        y = y * (1.5 - 0.5 * t * y * y)
    return y


def _sc_edge(edge_flat, zeros_p):
    mesh = plsc.VectorSubcoreMesh(
        core_axis_name="c", subcore_axis_name="s",
        num_cores=_NCORE, num_subcores=_NSUB)

    @functools.partial(
        pl.kernel,
        out_type=(jax.ShapeDtypeStruct((_P,), jnp.float32),
                  jax.ShapeDtypeStruct((_P,), jnp.float32),
                  jax.ShapeDtypeStruct((_P,), jnp.float32)),
        mesh=mesh,
        scratch_types=[
            pltpu.VMEM((_EPT,), jnp.int32),
            pltpu.VMEM((_EPT2,), jnp.int32),
            pltpu.VMEM((_P,), jnp.float32),
            pltpu.VMEM((_P,), jnp.float32),
            pltpu.VMEM((_NSUB, _SLAB), jnp.float32),
            pltpu.VMEM((_SLAB,), jnp.float32),
            pltpu.VMEM_SHARED((_NSUB, _P), jnp.float32),
            pltpu.VMEM_SHARED((_P,), jnp.float32),
        ],
        compiler_params=pltpu.CompilerParams(needs_layout_passes=False),
    )
    def k(edge_hbm, zero_hbm, dinv_hbm, s0_hbm, s1_hbm,
          dst_v, src2_v, acc_v, dinv_v, slab_v, co_v, part_sh, dinv_sh):
        cid = lax.axis_index("c")
        sid = lax.axis_index("s")
        base = sid * _EPT
        pltpu.sync_copy(edge_hbm.at[pl.ds(_E + base, _EPT)], dst_v)
        pltpu.sync_copy(zero_hbm, acc_v)
        ones = jnp.ones((16,), jnp.float32)

        def p1(i, c):
            idx = dst_v[pl.ds(i * 16, 16)]
            plsc.addupdate_scatter(acc_v, [idx], ones)
            return c
        lax.fori_loop(0, _NEV, p1, 0, unroll=8)

        pltpu.sync_copy(acc_v, part_sh.at[sid])
        plsc.subcore_barrier()
        pltpu.sync_copy(part_sh.at[:, pl.ds(sid * _SLAB, _SLAB)], slab_v)

        def rd(j, c):
            t = jnp.ones((16,), jnp.float32)
            for r in range(_NSUB):
                t = t + slab_v[r, pl.ds(j * 16, 16)]
            co_v[pl.ds(j * 16, 16)] = _fast_rsqrt(t)
            return c
        lax.fori_loop(0, _NSV, rd, 0)

        pltpu.sync_copy(co_v, dinv_sh.at[pl.ds(sid * _SLAB, _SLAB)])

        @pl.when(cid == 0)
        def _():
            pltpu.sync_copy(co_v, dinv_hbm.at[pl.ds(sid * _SLAB, _SLAB)])

        plsc.subcore_barrier()
        pltpu.sync_copy(dinv_sh, dinv_v)
        pltpu.sync_copy(zero_hbm, acc_v)

        half = cid * _EPT2
        pltpu.sync_copy(edge_hbm.at[pl.ds(base + half, _EPT2)], src2_v)

        def p2(i, c):
            di = dst_v[pl.ds(half + i * 16, 16)]
            vals = plsc.load_gather(dinv_v, [di])
            si = src2_v[pl.ds(i * 16, 16)]
            plsc.addupdate_scatter(acc_v, [si], vals)
            return c
        lax.fori_loop(0, _NEV2, p2, 0, unroll=8)

        pltpu.sync_copy(acc_v, part_sh.at[sid])
        plsc.subcore_barrier()
        pltpu.sync_copy(part_sh.at[:, pl.ds(sid * _SLAB, _SLAB)], slab_v)

        def rs(j, c):
            t = jnp.zeros((16,), jnp.float32)
            for r in range(_NSUB):
                t = t + slab_v[r, pl.ds(j * 16, 16)]
            co_v[pl.ds(j * 16, 16)] = t
            return c
        lax.fori_loop(0, _NSV, rs, 0)

        @pl.when(cid == 0)
        def _():
            pltpu.sync_copy(co_v, s0_hbm.at[pl.ds(sid * _SLAB, _SLAB)])

        @pl.when(cid == 1)
        def _():
            pltpu.sync_copy(co_v, s1_hbm.at[pl.ds(sid * _SLAB, _SLAB)])

    return k(edge_flat, zeros_p)


def _tc_evolve(x, pool_w2, wih, whh, bih2, bhh2, init_w):
    def body(x_ref, pw_ref, wih_ref, whh_ref, bih_ref, bhh_ref,
             h_ref, w_out_ref, sc_ref, xt_ref):
        pw = pw_ref[...]
        inv = 1.0 / jnp.sqrt(jnp.sum(pw * pw))

        def sc_loop(r, c):
            chunk = x_ref[pl.ds(r * 128, 128), :]
            srow = lax.dot_general(
                pw, chunk, (((1,), (1,)), ((), ())),
                preferred_element_type=jnp.float32)
            sc_ref[pl.ds(r, 1), :] = jnp.tanh(srow * inv)
            return c
        lax.fori_loop(0, _RF, sc_loop, 0)

        tail = x_ref[pl.ds(_RF * 128, 16), :]
        st = lax.dot_general(pw, tail, (((1,), (1,)), ((), ())),
                             preferred_element_type=jnp.float32)
        sc_ref[pl.ds(_RF, 1), :] = jnp.concatenate(
            [jnp.tanh(st * inv), jnp.full((1, 112), -2.0, jnp.float32)],
            axis=1)

        rows = lax.broadcasted_iota(jnp.int32, (_RS, 128), 0)
        cols = lax.broadcasted_iota(jnp.int32, (_RS, 128), 1)
        flat = rows * 128 + cols

        def pick(i, sw):
            m = jnp.max(sw)
            idx = jnp.min(jnp.where(sw == m, flat, jnp.int32(_P)))
            row = x_ref[pl.ds(idx, 1), :]
            xt_ref[pl.ds(i, 1), :] = row * m
            return jnp.where(flat == idx, -3.0, sw)
        lax.fori_loop(0, 128, pick, sc_ref[...])

        xt = xt_ref[...]
        gx = lax.dot_general(xt, wih_ref[...], (((1,), (1,)), ((), ())),
                             preferred_element_type=jnp.float32) + bih_ref[...]
        h = h_ref[...]
        gh = lax.dot_general(h, whh_ref[...], (((1,), (1,)), ((), ())),
                             preferred_element_type=jnp.float32) + bhh_ref[...]
        xr, xz, xn = gx[:, :128], gx[:, 128:256], gx[:, 256:]
        hr, hz, hn = gh[:, :128], gh[:, 128:256], gh[:, 256:]
        r_ = jax.nn.sigmoid(xr + hr)
        z_ = jax.nn.sigmoid(xz + hz)
        ng = jnp.tanh(xn + r_ * hn)
        w_out_ref[...] = (1.0 - z_) * ng + z_ * h

    return pl.pallas_call(
        body,
        out_shape=jax.ShapeDtypeStruct((128, 128), jnp.float32),
        scratch_shapes=[
            pltpu.VMEM((_RS, 128), jnp.float32),
            pltpu.VMEM((128, 128), jnp.float32),
        ],
    )(x, pool_w2, wih, whh, bih2, bhh2, init_w)


def _tc_head(dinv2, s2a, s2b, x_pad, w_g, gw_t, gb2, doc2, dw_t, db2,
             lg2, lb2, fw_t, fb2, tw_t, tb2, mw_t, mb2):
    def body(di_ref, sa_ref, sb_ref, xp_ref, w_ref, gw_ref, gb_ref, d_ref,
             dw_ref, db_ref, lg_ref, lb_ref, fw_ref, fb_ref, tw_ref, tb_ref,
             mw_ref, mb_ref, o1_ref, o2_ref):
        dv = di_ref[...]
        coeff = dv * (sa_ref[...] + sb_ref[...] + dv)
        cx = jnp.dot(coeff, xp_ref[...],
                     preferred_element_type=jnp.float32)
        pooled = jnp.dot(cx, w_ref[...],
                         preferred_element_type=jnp.float32) * (1.0 / _N)
        x_g = jnp.dot(pooled, gw_ref[...],
                      preferred_element_type=jnp.float32) + gb_ref[...]
        dq = jnp.maximum(
            jnp.dot(d_ref[...], dw_ref[...],
                    preferred_element_type=jnp.float32) + db_ref[...], 0.0)
        mu = (jnp.sum(x_g) + jnp.sum(dq)) / 512.0
        var = (jnp.sum((x_g - mu) ** 2) + jnp.sum((dq - mu) ** 2)) / 512.0
        isd = lax.rsqrt(var + 1e-5)
        a = (x_g - mu) * isd * lg_ref[:, :256] + lb_ref[:, :256]
        b = (dq - mu) * isd * lg_ref[:, 256:] + lb_ref[:, 256:]
        fused = (jnp.dot(a, fw_ref[:256, :], preferred_element_type=jnp.float32)
                 + jnp.dot(b, fw_ref[256:, :], preferred_element_type=jnp.float32)
                 + fb_ref[...])
        hh = jnp.maximum(fused, 0.0)
        o1_ref[...] = jnp.dot(hh, tw_ref[...],
                              preferred_element_type=jnp.float32) + tb_ref[...]
        o2_ref[...] = jnp.dot(hh, mw_ref[...],
                              preferred_element_type=jnp.float32) + mb_ref[...]

    return pl.pallas_call(
        body,
        out_shape=(jax.ShapeDtypeStruct((1, 10), jnp.float32),
                   jax.ShapeDtypeStruct((1, 1), jnp.float32)),
    )(dinv2, s2a, s2b, x_pad, w_g, gw_t, gb2, doc2, dw_t, db2,
      lg2, lb2, fw_t, fb2, tw_t, tb2, mw_t, mb2)


def kernel(x, edge_index, doc_feature, pool_w, gru_w_ih, gru_w_hh, gru_b_ih,
           gru_b_hh, init_w, gnn_fc_w, gnn_fc_b, doc_fc_w, doc_fc_b, ln_g,
           ln_b, fusion_w, fusion_b, task_w, task_b, time_w, time_b):
    zeros_p = jnp.zeros((_P,), jnp.float32)
    dinv, s0, s1 = _sc_edge(edge_index.reshape(2 * _E), zeros_p)

    w_g = _tc_evolve(
        x, pool_w.reshape(1, 128), gru_w_ih, gru_w_hh,
        gru_b_ih.reshape(1, -1), gru_b_hh.reshape(1, -1), init_w)

    x_pad = jnp.concatenate(
        [x, jnp.zeros((_P - _N, 128), jnp.float32)], axis=0)

    return _tc_head(
        dinv.reshape(1, _P), s0.reshape(1, _P), s1.reshape(1, _P),
        x_pad, w_g, gnn_fc_w.T, gnn_fc_b.reshape(1, -1),
        doc_feature.reshape(1, -1), doc_fc_w.T, doc_fc_b.reshape(1, -1),
        ln_g.reshape(1, -1), ln_b.reshape(1, -1), fusion_w.T,
        fusion_b.reshape(1, -1), task_w.T, task_b.reshape(1, -1),
        time_w.T, time_b.reshape(1, -1))


# speedup vs baseline: 100.8639x; 1.2453x over previous
import functools

import jax
import jax.numpy as jnp
from jax import lax
from jax.experimental import pallas as pl
from jax.experimental.pallas import tpu as pltpu
from jax.experimental.pallas import tpu_sc as plsc

_N = 10000
_E = 320000
_P = 10240
_NCORE = 2
_NSUB = 16
_EPT = _E // _NSUB
_NEV = _EPT // 16
_EPT2 = _E // (_NCORE * _NSUB)
_NEV2 = _EPT2 // 16
_SLAB = _P // _NSUB
_NSV = _SLAB // 16
_RF = 78
_RS = 79


def _fast_rsqrt(t):
    i = plsc.bitcast(t, jnp.int32)
    i = jnp.int32(0x5F3759DF) - lax.shift_right_logical(i, 1)
    y = plsc.bitcast(i, jnp.float32)
    for _ in range(3):
        y = y * (1.5 - 0.5 * t * y * y)
    return y


def _sc_edge(edge_flat, zeros_p):
    mesh = plsc.VectorSubcoreMesh(
        core_axis_name="c", subcore_axis_name="s",
        num_cores=_NCORE, num_subcores=_NSUB)

    @functools.partial(
        pl.kernel,
        out_type=(jax.ShapeDtypeStruct((_P,), jnp.float32),
                  jax.ShapeDtypeStruct((_P,), jnp.float32),
                  jax.ShapeDtypeStruct((_P,), jnp.float32)),
        mesh=mesh,
        scratch_types=[
            pltpu.VMEM((_EPT,), jnp.int32),
            pltpu.VMEM((_EPT2,), jnp.int32),
            pltpu.VMEM((_P,), jnp.float32),
            pltpu.VMEM((_P,), jnp.float32),
            pltpu.VMEM((_NSUB, _SLAB), jnp.float32),
            pltpu.VMEM((_SLAB,), jnp.float32),
            pltpu.VMEM_SHARED((_NSUB, _P), jnp.float32),
            pltpu.VMEM_SHARED((_P,), jnp.float32),
        ],
        compiler_params=pltpu.CompilerParams(needs_layout_passes=False),
    )
    def k(edge_hbm, zero_hbm, dinv_hbm, s0_hbm, s1_hbm,
          dst_v, src2_v, acc_v, dinv_v, slab_v, co_v, part_sh, dinv_sh):
        cid = lax.axis_index("c")
        sid = lax.axis_index("s")
        base = sid * _EPT
        pltpu.sync_copy(edge_hbm.at[pl.ds(_E + base, _EPT)], dst_v)
        pltpu.sync_copy(zero_hbm, acc_v)
        ones = jnp.ones((16,), jnp.float32)

        def p1(i, c):
            idx = dst_v[pl.ds(i * 16, 16)]
            plsc.addupdate_scatter(acc_v, [idx], ones)
            return c
        lax.fori_loop(0, _NEV, p1, 0, unroll=8)

        pltpu.sync_copy(acc_v, part_sh.at[sid])
        plsc.subcore_barrier()
        pltpu.sync_copy(part_sh.at[:, pl.ds(sid * _SLAB, _SLAB)], slab_v)

        def rd(j, c):
            t = jnp.ones((16,), jnp.float32)
            for r in range(_NSUB):
                t = t + slab_v[r, pl.ds(j * 16, 16)]
            co_v[pl.ds(j * 16, 16)] = _fast_rsqrt(t)
            return c
        lax.fori_loop(0, _NSV, rd, 0)

        pltpu.sync_copy(co_v, dinv_sh.at[pl.ds(sid * _SLAB, _SLAB)])

        @pl.when(cid == 0)
        def _():
            pltpu.sync_copy(co_v, dinv_hbm.at[pl.ds(sid * _SLAB, _SLAB)])

        plsc.subcore_barrier()
        pltpu.sync_copy(dinv_sh, dinv_v)
        pltpu.sync_copy(zero_hbm, acc_v)

        half = cid * _EPT2
        pltpu.sync_copy(edge_hbm.at[pl.ds(base + half, _EPT2)], src2_v)

        def p2(i, c):
            di = dst_v[pl.ds(half + i * 16, 16)]
            vals = plsc.load_gather(dinv_v, [di])
            si = src2_v[pl.ds(i * 16, 16)]
            plsc.addupdate_scatter(acc_v, [si], vals)
            return c
        lax.fori_loop(0, _NEV2, p2, 0, unroll=8)

        pltpu.sync_copy(acc_v, part_sh.at[sid])
        plsc.subcore_barrier()
        pltpu.sync_copy(part_sh.at[:, pl.ds(sid * _SLAB, _SLAB)], slab_v)

        def rs(j, c):
            t = jnp.zeros((16,), jnp.float32)
            for r in range(_NSUB):
                t = t + slab_v[r, pl.ds(j * 16, 16)]
            co_v[pl.ds(j * 16, 16)] = t
            return c
        lax.fori_loop(0, _NSV, rs, 0)

        @pl.when(cid == 0)
        def _():
            pltpu.sync_copy(co_v, s0_hbm.at[pl.ds(sid * _SLAB, _SLAB)])

        @pl.when(cid == 1)
        def _():
            pltpu.sync_copy(co_v, s1_hbm.at[pl.ds(sid * _SLAB, _SLAB)])

    return k(edge_flat, zeros_p)


def _tc_evolve(x, pool_w2, wih, whh, bih2, bhh2, init_w):
    def body(x_ref, pw_ref, wih_ref, whh_ref, bih_ref, bhh_ref,
             h_ref, w_out_ref, sc_ref, xt_ref):
        pw = pw_ref[...]
        inv = 1.0 / jnp.sqrt(jnp.sum(pw * pw))

        def sc_loop(r, c):
            chunk = x_ref[pl.ds(r * 128, 128), :]
            srow = lax.dot_general(
                pw, chunk, (((1,), (1,)), ((), ())),
                preferred_element_type=jnp.float32)
            sc_ref[pl.ds(r, 1), :] = srow
            return c
        lax.fori_loop(0, _RF, sc_loop, 0, unroll=6)

        tail = x_ref[pl.ds(_RF * 128, 16), :]
        st = lax.dot_general(pw, tail, (((1,), (1,)), ((), ())),
                             preferred_element_type=jnp.float32)
        sc_ref[pl.ds(_RF, 1), :] = jnp.concatenate(
            [st, jnp.full((1, 112), -1e30, jnp.float32)], axis=1)

        rows = lax.broadcasted_iota(jnp.int32, (_RS, 128), 0)
        cols = lax.broadcasted_iota(jnp.int32, (_RS, 128), 1)
        flatf = (rows * 128 + cols).astype(jnp.float32)

        sw0 = jnp.where(flatf < jnp.float32(_N), sc_ref[...], -1e30)

        def pick(i, sw):
            m = jnp.max(sw)
            idxf = jnp.min(jnp.where(sw == m, flatf, jnp.float32(1e9)))
            idx = idxf.astype(jnp.int32)
            row = x_ref[pl.ds(idx, 1), :]
            xt_ref[pl.ds(i, 1), :] = row * jnp.tanh(m * inv)
            return jnp.where(flatf == idxf, -2e30, sw)
        lax.fori_loop(0, 128, pick, sw0)

        xt = xt_ref[...]
        gx = lax.dot_general(xt, wih_ref[...], (((1,), (1,)), ((), ())),
                             preferred_element_type=jnp.float32) + bih_ref[...]
        h = h_ref[...]
        gh = lax.dot_general(h, whh_ref[...], (((1,), (1,)), ((), ())),
                             preferred_element_type=jnp.float32) + bhh_ref[...]
        xr, xz, xn = gx[:, :128], gx[:, 128:256], gx[:, 256:]
        hr, hz, hn = gh[:, :128], gh[:, 128:256], gh[:, 256:]
        r_ = jax.nn.sigmoid(xr + hr)
        z_ = jax.nn.sigmoid(xz + hz)
        ng = jnp.tanh(xn + r_ * hn)
        w_out_ref[...] = (1.0 - z_) * ng + z_ * h

    return pl.pallas_call(
        body,
        out_shape=jax.ShapeDtypeStruct((128, 128), jnp.float32),
        scratch_shapes=[
            pltpu.VMEM((_RS, 128), jnp.float32),
            pltpu.VMEM((128, 128), jnp.float32),
        ],
    )(x, pool_w2, wih, whh, bih2, bhh2, init_w)


def _tc_head(dinv2, s2a, s2b, x_pad, w_g, gw_t, gb2, doc2, dw_t, db2,
             lg2, lb2, fw_t, fb2, tw_t, tb2, mw_t, mb2):
    def body(di_ref, sa_ref, sb_ref, xp_ref, w_ref, gw_ref, gb_ref, d_ref,
             dw_ref, db_ref, lg_ref, lb_ref, fw_ref, fb_ref, tw_ref, tb_ref,
             mw_ref, mb_ref, o1_ref, o2_ref):
        dv = di_ref[...]
        coeff = dv * (sa_ref[...] + sb_ref[...] + dv)
        cx = jnp.dot(coeff, xp_ref[...],
                     preferred_element_type=jnp.float32)
        pooled = jnp.dot(cx, w_ref[...],
                         preferred_element_type=jnp.float32) * (1.0 / _N)
        x_g = jnp.dot(pooled, gw_ref[...],
                      preferred_element_type=jnp.float32) + gb_ref[...]
        dq = jnp.maximum(
            jnp.dot(d_ref[...], dw_ref[...],
                    preferred_element_type=jnp.float32) + db_ref[...], 0.0)
        mu = (jnp.sum(x_g) + jnp.sum(dq)) / 512.0
        var = (jnp.sum((x_g - mu) ** 2) + jnp.sum((dq - mu) ** 2)) / 512.0
        isd = lax.rsqrt(var + 1e-5)
        a = (x_g - mu) * isd * lg_ref[:, :256] + lb_ref[:, :256]
        b = (dq - mu) * isd * lg_ref[:, 256:] + lb_ref[:, 256:]
        fused = (jnp.dot(a, fw_ref[:256, :], preferred_element_type=jnp.float32)
                 + jnp.dot(b, fw_ref[256:, :], preferred_element_type=jnp.float32)
                 + fb_ref[...])
        hh = jnp.maximum(fused, 0.0)
        o1_ref[...] = jnp.dot(hh, tw_ref[...],
                              preferred_element_type=jnp.float32) + tb_ref[...]
        o2_ref[...] = jnp.dot(hh, mw_ref[...],
                              preferred_element_type=jnp.float32) + mb_ref[...]

    return pl.pallas_call(
        body,
        out_shape=(jax.ShapeDtypeStruct((1, 10), jnp.float32),
                   jax.ShapeDtypeStruct((1, 1), jnp.float32)),
    )(dinv2, s2a, s2b, x_pad, w_g, gw_t, gb2, doc2, dw_t, db2,
      lg2, lb2, fw_t, fb2, tw_t, tb2, mw_t, mb2)


def kernel(x, edge_index, doc_feature, pool_w, gru_w_ih, gru_w_hh, gru_b_ih,
           gru_b_hh, init_w, gnn_fc_w, gnn_fc_b, doc_fc_w, doc_fc_b, ln_g,
           ln_b, fusion_w, fusion_b, task_w, task_b, time_w, time_b):
    zeros_p = jnp.zeros((_P,), jnp.float32)
    dinv, s0, s1 = _sc_edge(edge_index.reshape(2 * _E), zeros_p)

    w_g = _tc_evolve(
        x, pool_w.reshape(1, 128), gru_w_ih, gru_w_hh,
        gru_b_ih.reshape(1, -1), gru_b_hh.reshape(1, -1), init_w)

    x_pad = jnp.concatenate(
        [x, jnp.zeros((_P - _N, 128), jnp.float32)], axis=0)

    return _tc_head(
        dinv.reshape(1, _P), s0.reshape(1, _P), s1.reshape(1, _P),
        x_pad, w_g, gnn_fc_w.T, gnn_fc_b.reshape(1, -1),
        doc_feature.reshape(1, -1), doc_fc_w.T, doc_fc_b.reshape(1, -1),
        ln_g.reshape(1, -1), ln_b.reshape(1, -1), fusion_w.T,
        fusion_b.reshape(1, -1), task_w.T, task_b.reshape(1, -1),
        time_w.T, time_b.reshape(1, -1))


# speedup vs baseline: 108.0803x; 1.0715x over previous
import functools

import jax
import jax.numpy as jnp
from jax import lax
from jax.experimental import pallas as pl
from jax.experimental.pallas import tpu as pltpu
from jax.experimental.pallas import tpu_sc as plsc

_N = 10000
_E = 320000
_P = 10240
_NCORE = 2
_NSUB = 16
_EPT = _E // _NSUB
_NEV = _EPT // 16
_EPT2 = _E // (_NCORE * _NSUB)
_NEV2 = _EPT2 // 16
_SLAB = _P // _NSUB
_NSV = _SLAB // 16
_RF = 78
_RS = 79


def _fast_rsqrt(t):
    i = plsc.bitcast(t, jnp.int32)
    i = jnp.int32(0x5F3759DF) - lax.shift_right_logical(i, 1)
    y = plsc.bitcast(i, jnp.float32)
    for _ in range(3):
        y = y * (1.5 - 0.5 * t * y * y)
    return y


def _sc_edge(edge_flat, zeros_p):
    mesh = plsc.VectorSubcoreMesh(
        core_axis_name="c", subcore_axis_name="s",
        num_cores=_NCORE, num_subcores=_NSUB)

    @functools.partial(
        pl.kernel,
        out_type=(jax.ShapeDtypeStruct((_P,), jnp.float32),
                  jax.ShapeDtypeStruct((_P,), jnp.float32),
                  jax.ShapeDtypeStruct((_P,), jnp.float32)),
        mesh=mesh,
        scratch_types=[
            pltpu.VMEM((_EPT,), jnp.int32),
            pltpu.VMEM((_EPT2,), jnp.int32),
            pltpu.VMEM((_P,), jnp.float32),
            pltpu.VMEM((_P,), jnp.float32),
            pltpu.VMEM((_NSUB, _SLAB), jnp.float32),
            pltpu.VMEM((_SLAB,), jnp.float32),
            pltpu.VMEM_SHARED((_NSUB, _P), jnp.float32),
            pltpu.VMEM_SHARED((_P,), jnp.float32),
        ],
        compiler_params=pltpu.CompilerParams(needs_layout_passes=False),
    )
    def k(edge_hbm, zero_hbm, dinv_hbm, s0_hbm, s1_hbm,
          dst_v, src2_v, acc_v, dinv_v, slab_v, co_v, part_sh, dinv_sh):
        cid = lax.axis_index("c")
        sid = lax.axis_index("s")
        base = sid * _EPT
        pltpu.sync_copy(edge_hbm.at[pl.ds(_E + base, _EPT)], dst_v)
        pltpu.sync_copy(zero_hbm, acc_v)
        ones = jnp.ones((16,), jnp.float32)

        def p1(i, c):
            idx = dst_v[pl.ds(i * 16, 16)]
            plsc.addupdate_scatter(acc_v, [idx], ones)
            return c
        lax.fori_loop(0, _NEV, p1, 0, unroll=8)

        pltpu.sync_copy(acc_v, part_sh.at[sid])
        plsc.subcore_barrier()
        pltpu.sync_copy(part_sh.at[:, pl.ds(sid * _SLAB, _SLAB)], slab_v)

        def rd(j, c):
            t = jnp.ones((16,), jnp.float32)
            for r in range(_NSUB):
                t = t + slab_v[r, pl.ds(j * 16, 16)]
            co_v[pl.ds(j * 16, 16)] = _fast_rsqrt(t)
            return c
        lax.fori_loop(0, _NSV, rd, 0)

        pltpu.sync_copy(co_v, dinv_sh.at[pl.ds(sid * _SLAB, _SLAB)])

        @pl.when(cid == 0)
        def _():
            pltpu.sync_copy(co_v, dinv_hbm.at[pl.ds(sid * _SLAB, _SLAB)])

        plsc.subcore_barrier()
        pltpu.sync_copy(dinv_sh, dinv_v)
        pltpu.sync_copy(zero_hbm, acc_v)

        half = cid * _EPT2
        pltpu.sync_copy(edge_hbm.at[pl.ds(base + half, _EPT2)], src2_v)

        def p2(i, c):
            di = dst_v[pl.ds(half + i * 16, 16)]
            vals = plsc.load_gather(dinv_v, [di])
            si = src2_v[pl.ds(i * 16, 16)]
            plsc.addupdate_scatter(acc_v, [si], vals)
            return c
        lax.fori_loop(0, _NEV2, p2, 0, unroll=8)

        pltpu.sync_copy(acc_v, part_sh.at[sid])
        plsc.subcore_barrier()
        pltpu.sync_copy(part_sh.at[:, pl.ds(sid * _SLAB, _SLAB)], slab_v)

        def rs(j, c):
            t = jnp.zeros((16,), jnp.float32)
            for r in range(_NSUB):
                t = t + slab_v[r, pl.ds(j * 16, 16)]
            co_v[pl.ds(j * 16, 16)] = t
            return c
        lax.fori_loop(0, _NSV, rs, 0)

        @pl.when(cid == 0)
        def _():
            pltpu.sync_copy(co_v, s0_hbm.at[pl.ds(sid * _SLAB, _SLAB)])

        @pl.when(cid == 1)
        def _():
            pltpu.sync_copy(co_v, s1_hbm.at[pl.ds(sid * _SLAB, _SLAB)])

    return k(edge_flat, zeros_p)


def _tc_evolve(x, pool_w2, wih, whh, bih2, bhh2, init_w):
    def body(x_ref, pw_ref, wih_ref, whh_ref, bih_ref, bhh_ref,
             h_ref, w_out_ref, sc_ref, xt_ref):
        pw = pw_ref[...]
        inv = 1.0 / jnp.sqrt(jnp.sum(pw * pw))

        def sc_loop(r, c):
            chunk = x_ref[pl.ds(r * 128, 128), :]
            srow = lax.dot_general(
                pw, chunk, (((1,), (1,)), ((), ())),
                preferred_element_type=jnp.float32)
            sc_ref[pl.ds(r, 1), :] = srow
            return c
        lax.fori_loop(0, _RF, sc_loop, 0, unroll=6)

        tail = x_ref[pl.ds(_RF * 128, 16), :]
        st = lax.dot_general(pw, tail, (((1,), (1,)), ((), ())),
                             preferred_element_type=jnp.float32)
        sc_ref[pl.ds(_RF, 1), :] = jnp.concatenate(
            [st, jnp.full((1, 112), -1e30, jnp.float32)], axis=1)

        rows = lax.broadcasted_iota(jnp.int32, (_RS, 128), 0)
        cols = lax.broadcasted_iota(jnp.int32, (_RS, 128), 1)
        flatf = (rows * 128 + cols).astype(jnp.float32)

        sw0 = jnp.where(flatf < jnp.float32(_N), sc_ref[...], -1e30)

        def pick(i, sw):
            m = jnp.max(sw)
            idxf = jnp.min(jnp.where(sw == m, flatf, jnp.float32(1e9)))
            idx = idxf.astype(jnp.int32)
            row = x_ref[pl.ds(idx, 1), :]
            xt_ref[pl.ds(i, 1), :] = row * jnp.tanh(m * inv)
            return jnp.where(flatf == idxf, -2e30, sw)
        lax.fori_loop(0, 128, pick, sw0)

        xt = xt_ref[...]
        gx = lax.dot_general(xt, wih_ref[...], (((1,), (1,)), ((), ())),
                             preferred_element_type=jnp.float32) + bih_ref[...]
        h = h_ref[...]
        gh = lax.dot_general(h, whh_ref[...], (((1,), (1,)), ((), ())),
                             preferred_element_type=jnp.float32) + bhh_ref[...]
        xr, xz, xn = gx[:, :128], gx[:, 128:256], gx[:, 256:]
        hr, hz, hn = gh[:, :128], gh[:, 128:256], gh[:, 256:]
        r_ = jax.nn.sigmoid(xr + hr)
        z_ = jax.nn.sigmoid(xz + hz)
        ng = jnp.tanh(xn + r_ * hn)
        w_out_ref[...] = (1.0 - z_) * ng + z_ * h

    return pl.pallas_call(
        body,
        out_shape=jax.ShapeDtypeStruct((128, 128), jnp.float32),
        scratch_shapes=[
            pltpu.VMEM((_RS, 128), jnp.float32),
            pltpu.VMEM((128, 128), jnp.float32),
        ],
    )(x, pool_w2, wih, whh, bih2, bhh2, init_w)


def _tc_head(dinv2, s2a, s2b, x_pad, w_g, gw_t, gb2, doc2, dw_t, db2,
             lg2, lb2, fw_t, fb2, tw_t, tb2, mw_t, mb2):
    def body(di_ref, sa_ref, sb_ref, xp_ref, w_ref, gw_ref, gb_ref, d_ref,
             dw_ref, db_ref, lg_ref, lb_ref, fw_ref, fb_ref, tw_ref, tb_ref,
             mw_ref, mb_ref, o1_ref, o2_ref):
        def nt(a, b):
            return lax.dot_general(a, b, (((1,), (1,)), ((), ())),
                                   preferred_element_type=jnp.float32)

        dv = di_ref[...]
        coeff = dv * (sa_ref[...] + sb_ref[...] + dv)
        cx = jnp.dot(coeff, xp_ref[...],
                     preferred_element_type=jnp.float32)
        pooled = jnp.dot(cx, w_ref[...],
                         preferred_element_type=jnp.float32) * (1.0 / _N)
        x_g = nt(pooled, gw_ref[...]) + gb_ref[...]
        dq = jnp.maximum(nt(d_ref[...], dw_ref[...]) + db_ref[...], 0.0)
        mu = (jnp.sum(x_g) + jnp.sum(dq)) / 512.0
        var = (jnp.sum((x_g - mu) ** 2) + jnp.sum((dq - mu) ** 2)) / 512.0
        isd = lax.rsqrt(var + 1e-5)
        a = (x_g - mu) * isd * lg_ref[:, :256] + lb_ref[:, :256]
        b = (dq - mu) * isd * lg_ref[:, 256:] + lb_ref[:, 256:]
        fused = nt(a, fw_ref[:, :256]) + nt(b, fw_ref[:, 256:]) + fb_ref[...]
        hh = jnp.maximum(fused, 0.0)
        o1_ref[...] = jnp.dot(hh, tw_ref[...],
                              preferred_element_type=jnp.float32) + tb_ref[...]
        o2_ref[...] = jnp.dot(hh, mw_ref[...],
                              preferred_element_type=jnp.float32) + mb_ref[...]

    return pl.pallas_call(
        body,
        out_shape=(jax.ShapeDtypeStruct((1, 10), jnp.float32),
                   jax.ShapeDtypeStruct((1, 1), jnp.float32)),
    )(dinv2, s2a, s2b, x_pad, w_g, gw_t, gb2, doc2, dw_t, db2,
      lg2, lb2, fw_t, fb2, tw_t, tb2, mw_t, mb2)


def kernel(x, edge_index, doc_feature, pool_w, gru_w_ih, gru_w_hh, gru_b_ih,
           gru_b_hh, init_w, gnn_fc_w, gnn_fc_b, doc_fc_w, doc_fc_b, ln_g,
           ln_b, fusion_w, fusion_b, task_w, task_b, time_w, time_b):
    zeros_p = jnp.zeros((_P,), jnp.float32)
    dinv, s0, s1 = _sc_edge(edge_index.reshape(2 * _E), zeros_p)

    w_g = _tc_evolve(
        x, pool_w.reshape(1, 128), gru_w_ih, gru_w_hh,
        gru_b_ih.reshape(1, -1), gru_b_hh.reshape(1, -1), init_w)

    x_pad = jnp.concatenate(
        [x, jnp.zeros((_P - _N, 128), jnp.float32)], axis=0)

    return _tc_head(
        dinv.reshape(1, _P), s0.reshape(1, _P), s1.reshape(1, _P),
        x_pad, w_g, gnn_fc_w, gnn_fc_b.reshape(1, -1),
        doc_feature.reshape(1, -1), doc_fc_w, doc_fc_b.reshape(1, -1),
        ln_g.reshape(1, -1), ln_b.reshape(1, -1), fusion_w,
        fusion_b.reshape(1, -1), task_w.T, task_b.reshape(1, -1),
        time_w.T, time_b.reshape(1, -1))
